# baseline (speedup 1.0000x reference)
import functools

import jax
import jax.numpy as jnp
from jax import lax
from jax.experimental import pallas as pl
from jax.experimental.pallas import tpu as pltpu
from jax.experimental.pallas import tpu_sc as plsc

_NC = 2
_NS = 16
_NW = _NC * _NS


def _table_body(embed_ref, lnw_ref, lnb_ref, wt_ref, bias_ref, out_ref):
    e = embed_ref[...]
    mean = jnp.mean(e, axis=1, keepdims=True)
    c = e - mean
    var = jnp.mean(c * c, axis=1, keepdims=True)
    h = (c / jnp.sqrt(var + 1e-5)) * lnw_ref[...][None, :] + lnb_ref[...][None, :]
    out_ref[...] = (
        jnp.dot(h, wt_ref[...], preferred_element_type=jnp.float32)
        + bias_ref[...][None, :]
    )


def _compute_table(embed, ln_w, ln_b, wt, bias):
    v = wt.shape[1]
    return pl.pallas_call(
        _table_body,
        out_shape=jax.ShapeDtypeStruct((embed.shape[0], v), jnp.float32),
    )(embed, ln_w, ln_b, wt, bias)


@functools.lru_cache(maxsize=None)
def _make_gather(n_tokens, v):
    b_per_w = n_tokens // _NW
    chunk = 64
    n_chunks = b_per_w // chunk
    mesh = plsc.VectorSubcoreMesh(core_axis_name="c", subcore_axis_name="s")

    @functools.partial(
        pl.kernel,
        mesh=mesh,
        compiler_params=pltpu.CompilerParams(use_tc_tiling_on_sc=False),
        out_type=jax.ShapeDtypeStruct((n_tokens, v), jnp.float32),
        scratch_types=[
            pltpu.VMEM((b_per_w,), jnp.int32),
            pltpu.VMEM((chunk, v), jnp.float32),
            pltpu.SemaphoreType.DMA,
        ],
    )
    def gather(table_hbm, idx_hbm, out_hbm, idx_v, rows_v, sem):
        wid = lax.axis_index("s") * _NC + lax.axis_index("c")
        base = wid * b_per_w
        pltpu.sync_copy(idx_hbm.at[pl.ds(base, b_per_w)], idx_v)

        def body(i, carry):
            off = i * chunk
            pltpu.async_copy(
                table_hbm.at[idx_v.at[pl.ds(off, chunk)]], rows_v, sem
            ).wait()
            pltpu.sync_copy(rows_v, out_hbm.at[pl.ds(base + off, chunk)])
            return carry

        lax.fori_loop(0, n_chunks, body, 0)

    return gather


def kernel(input_ids, embed, ln_w, ln_b, W, b):
    bsz, seq = input_ids.shape
    vocab = W.shape[0]
    table = _compute_table(embed, ln_w, ln_b, W.T, b)
    ids = input_ids.reshape(-1).astype(jnp.int32)
    out_flat = _make_gather(bsz * seq, vocab)(table, ids)
    return out_flat.reshape(bsz, seq, vocab)


# speedup vs baseline: 1.0212x; 1.0212x over previous
import functools

import jax
import jax.numpy as jnp
from jax import lax
from jax.experimental import pallas as pl
from jax.experimental.pallas import tpu as pltpu
from jax.experimental.pallas import tpu_sc as plsc

_NC = 2
_NS = 16
_NW = _NC * _NS


def _table_body(embed_ref, lnw_ref, lnb_ref, wt_ref, bias_ref, out_ref):
    e = embed_ref[...]
    mean = jnp.mean(e, axis=1, keepdims=True)
    c = e - mean
    var = jnp.mean(c * c, axis=1, keepdims=True)
    h = (c / jnp.sqrt(var + 1e-5)) * lnw_ref[...][None, :] + lnb_ref[...][None, :]
    out_ref[...] = (
        jnp.dot(h, wt_ref[...], preferred_element_type=jnp.float32)
        + bias_ref[...][None, :]
    )


def _compute_table(embed, ln_w, ln_b, wt, bias):
    v = wt.shape[1]
    return pl.pallas_call(
        _table_body,
        out_shape=jax.ShapeDtypeStruct((embed.shape[0], v), jnp.float32),
    )(embed, ln_w, ln_b, wt, bias)


@functools.lru_cache(maxsize=None)
def _make_gather(n_tokens, v):
    b_per_w = n_tokens // _NW
    chunk = 40
    n_chunks = b_per_w // chunk
    mesh = plsc.VectorSubcoreMesh(core_axis_name="c", subcore_axis_name="s")

    @functools.partial(
        pl.kernel,
        mesh=mesh,
        compiler_params=pltpu.CompilerParams(use_tc_tiling_on_sc=False),
        out_type=jax.ShapeDtypeStruct((n_tokens, v), jnp.float32),
        scratch_types=[
            pltpu.VMEM((b_per_w,), jnp.int32),
            pltpu.VMEM((chunk, v), jnp.float32),
            pltpu.VMEM((chunk, v), jnp.float32),
            pltpu.SemaphoreType.DMA,
            pltpu.SemaphoreType.DMA,
            pltpu.SemaphoreType.DMA,
            pltpu.SemaphoreType.DMA,
        ],
    )
    def gather(table_hbm, idx_hbm, out_hbm, idx_v, rows0, rows1, g0, g1, w0, w1):
        wid = lax.axis_index("s") * _NC + lax.axis_index("c")
        base = wid * b_per_w
        pltpu.sync_copy(idx_hbm.at[pl.ds(base, b_per_w)], idx_v)
        bufs = (rows0, rows1)
        gsem = (g0, g1)
        wsem = (w0, w1)

        def start_gather(i, b):
            pltpu.async_copy(
                table_hbm.at[idx_v.at[pl.ds(i * chunk, chunk)]], bufs[b], gsem[b]
            )

        def start_write(i, b):
            pltpu.async_copy(bufs[b], out_hbm.at[pl.ds(base + i * chunk, chunk)], wsem[b])

        start_gather(0, 0)

        def body(g, carry):
            for bsel in range(2):
                i = g * 2 + bsel
                cur, nxt = bsel, 1 - bsel
                pltpu.make_async_copy(table_hbm.at[idx_v.at[pl.ds(0, chunk)]],
                                      bufs[cur], gsem[cur]).wait()

                @pl.when(i >= 1)
                def _():
                    pltpu.make_async_copy(
                        bufs[nxt], out_hbm.at[pl.ds(base, chunk)], wsem[nxt]
                    ).wait()

                @pl.when(i + 1 < n_chunks)
                def _():
                    start_gather(i + 1, nxt)

                start_write(i, cur)
            return carry

        lax.fori_loop(0, n_chunks // 2, body, 0)
        pltpu.make_async_copy(bufs[1], out_hbm.at[pl.ds(base, chunk)], wsem[1]).wait()

    return gather


def kernel(input_ids, embed, ln_w, ln_b, W, b):
    bsz, seq = input_ids.shape
    vocab = W.shape[0]
    table = _compute_table(embed, ln_w, ln_b, W.T, b)
    ids = input_ids.reshape(-1).astype(jnp.int32)
    out_flat = _make_gather(bsz * seq, vocab)(table, ids)
    return out_flat.reshape(bsz, seq, vocab)
